# baseline (speedup 1.0000x reference)
import jax
import jax.numpy as jnp
from jax.experimental import pallas as pl

DIM = 64
N_EMBED = 1024
PIX = 1024
BPS = 4


def _vq_body(z_ref, e_ref, zq_ref, idx_ref):
    emb = e_ref[...]
    e2 = jnp.sum(emb * emb, axis=1, keepdims=True)
    emb_m2 = -2.0 * emb
    iota_n = jax.lax.broadcasted_iota(jnp.int32, (N_EMBED, PIX), 0)
    nvec = jax.lax.broadcasted_iota(jnp.int32, (8, N_EMBED), 1).astype(jnp.float32)
    w_hi = jnp.floor(nvec / 4.0)
    w_lo = nvec - 4.0 * w_hi
    row = jax.lax.broadcasted_iota(jnp.int32, (8, N_EMBED), 0)
    wmat = jnp.where(row == 0, w_hi,
                     jnp.where(row == 1, w_lo,
                               jnp.where(row == 2, 1.0, 0.0)))
    for j in range(BPS):
        zb = z_ref[j]
        s_m2 = jax.lax.dot_general(
            emb_m2, zb, (((1,), (0,)), ((), ())),
            preferred_element_type=jnp.float32,
            precision=jax.lax.Precision.DEFAULT)
        z2 = jnp.sum(zb * zb, axis=0, keepdims=True)
        d = (z2 + e2) + s_m2
        m = jnp.min(d, axis=0, keepdims=True)
        maskf = jnp.where(d == m, 1.0, 0.0)
        ext = jax.lax.dot_general(
            wmat, maskf, (((1,), (0,)), ((), ())),
            preferred_element_type=jnp.float32,
            precision=jax.lax.Precision.DEFAULT)
        idx = (4.0 * ext[0] + ext[1]).astype(jnp.int32)
        idx_ref[j, 0] = idx
        zq_ref[j] = jax.lax.dot_general(
            emb, maskf, (((0,), (0,)), ((), ())),
            preferred_element_type=jnp.float32,
            precision=jax.lax.Precision.DEFAULT)

        @pl.when(jnp.max(ext[2]) > 1.5)
        def _repair():
            cand = jnp.where(maskf > 0.5, iota_n.astype(jnp.float32), 3e38)
            idx2 = jnp.min(cand, axis=0).astype(jnp.int32)
            idx_ref[j, 0] = idx2
            onehot = (iota_n == idx2[None, :]).astype(jnp.float32)
            zq_ref[j] = jax.lax.dot_general(
                emb, onehot, (((0,), (0,)), ((), ())),
                preferred_element_type=jnp.float32,
                precision=jax.lax.Precision.DEFAULT)


def kernel(z, embedding):
    b = z.shape[0]
    z3 = z.reshape(b, DIM, PIX)
    zq, idx = pl.pallas_call(
        _vq_body,
        grid=(b // BPS,),
        in_specs=[
            pl.BlockSpec((BPS, DIM, PIX), lambda i: (i, 0, 0)),
            pl.BlockSpec((N_EMBED, DIM), lambda i: (0, 0)),
        ],
        out_specs=[
            pl.BlockSpec((BPS, DIM, PIX), lambda i: (i, 0, 0)),
            pl.BlockSpec((BPS, 1, PIX), lambda i: (i, 0, 0)),
        ],
        out_shape=[
            jax.ShapeDtypeStruct((b, DIM, PIX), jnp.float32),
            jax.ShapeDtypeStruct((b, 1, PIX), jnp.int32),
        ],
    )(z3, embedding)
    return zq.reshape(z.shape), idx.reshape(b * PIX)
